# Initial kernel scaffold; baseline (speedup 1.0000x reference)
import jax
import jax.numpy as jnp
from jax.experimental import pallas as pl


def kernel(pos, feat, W1, b1, g1, be1, W2, b2, g2, be2, W3, b3, g3, be3):
    raise NotImplementedError("write your pallas kernel here")


# speedup vs baseline: 5.6943x; 5.6943x over previous
import functools

import jax
import jax.numpy as jnp
from jax import lax
from jax.experimental import pallas as pl
from jax.experimental.pallas import tpu as pltpu
from jax.experimental.pallas import tpu_sc as plsc

_P = 1024
_K = 64
_EPS = 1e-5
_HIGH = lax.Precision.HIGHEST
_D = 128
_PT = 128
_GT = 32
_BIGF = 3.0e38


def _fps_body(xyz_ref, cpx_ref, cpy_ref, cpz_ref):
    X = xyz_ref[0]
    Y = xyz_ref[1]
    Z = xyz_ref[2]
    B, N = X.shape
    iota_n = lax.broadcasted_iota(jnp.int32, (B, N), 1)
    iota_p = lax.broadcasted_iota(jnp.int32, (B, _P), 1)

    def body(i, st):
        dist, far, cpx, cpy, cpz = st
        sel = iota_n == far
        cx = jnp.sum(jnp.where(sel, X, 0.0), axis=1, keepdims=True)
        cy = jnp.sum(jnp.where(sel, Y, 0.0), axis=1, keepdims=True)
        cz = jnp.sum(jnp.where(sel, Z, 0.0), axis=1, keepdims=True)
        col = iota_p == i
        cpx = jnp.where(col, cx, cpx)
        cpy = jnp.where(col, cy, cpy)
        cpz = jnp.where(col, cz, cpz)
        dx = X - cx
        dy = Y - cy
        dz = Z - cz
        d = dx * dx + (dy * dy + dz * dz)
        dist = jnp.minimum(dist, d)
        m = jnp.max(dist, axis=1, keepdims=True)
        far = jnp.max(jnp.where(dist == m, iota_n, -1), axis=1, keepdims=True)
        return dist, far, cpx, cpy, cpz

    st0 = (
        jnp.full((B, N), 1e10, jnp.float32),
        jnp.zeros((B, 1), jnp.int32),
        jnp.zeros((B, _P), jnp.float32),
        jnp.zeros((B, _P), jnp.float32),
        jnp.zeros((B, _P), jnp.float32),
    )
    _, _, cpx, cpy, cpz = lax.fori_loop(0, _P, body, st0)
    cpx_ref[...] = cpx
    cpy_ref[...] = cpy
    cpz_ref[...] = cpz


def _run_fps(pos):
    B, N, _ = pos.shape
    pos_s = jnp.transpose(pos, (2, 0, 1))
    out = jax.ShapeDtypeStruct((B, _P), jnp.float32)
    cpx, cpy, cpz = pl.pallas_call(
        _fps_body,
        out_shape=[out, out, out],
    )(pos_s)
    return jnp.stack([cpx, cpy, cpz], axis=-1)


def _knn_body(posr_ref, cen_ref, out_ref):
    b = pl.program_id(0)
    N = posr_ref.shape[2]
    px = posr_ref[0, 0:1, :]
    py = posr_ref[0, 1:2, :]
    pz = posr_ref[0, 2:3, :]
    cx = cen_ref[0, :, 0:1]
    cy = cen_ref[0, :, 1:2]
    cz = cen_ref[0, :, 2:3]
    dot = jnp.dot(cen_ref[0], posr_ref[0],
                  preferred_element_type=jnp.float32)
    cn = (cx * cx + cy * cy) + cz * cz
    pn = (px * px + py * py) + pz * pz
    d = (-2.0 * dot + cn) + pn
    iota = lax.broadcasted_iota(jnp.int32, (1, N), 1)
    base = b * N
    for j in range(_K):
        m = jnp.min(d, axis=1, keepdims=True)
        idx = jnp.min(jnp.where(d == m, iota, N), axis=1, keepdims=True)
        out_ref[0, :, pl.ds(j, 1)] = idx + base
        d = jnp.where(iota == idx, _BIGF, d)


def _run_knn(pos, center_pos):
    B, N, _ = pos.shape
    pos_r = jnp.transpose(pos, (0, 2, 1))
    grid = (B, _P // _PT)
    idx = pl.pallas_call(
        _knn_body,
        grid=grid,
        in_specs=[
            pl.BlockSpec((1, 3, N), lambda b, t: (b, 0, 0)),
            pl.BlockSpec((1, _PT, 3), lambda b, t: (b, t, 0)),
        ],
        out_specs=pl.BlockSpec((1, _PT, _K), lambda b, t: (b, t, 0)),
        out_shape=jax.ShapeDtypeStruct((B, _P, _K), jnp.int32),
    )(pos_r, center_pos)
    return idx.reshape(B * _P * _K)


def _run_gather(table, idxf):
    M = idxf.shape[0]
    info = plsc.get_sparse_core_info()
    nw = info.num_cores * info.num_subcores
    mw = M // nw
    ch = 128
    mesh = plsc.VectorSubcoreMesh(core_axis_name="c", subcore_axis_name="s")

    @functools.partial(
        pl.kernel,
        out_type=jax.ShapeDtypeStruct((M, _D), jnp.float32),
        mesh=mesh,
        scratch_types=[
            pltpu.VMEM((ch,), jnp.int32),
            pltpu.VMEM((ch, _D), jnp.float32),
            pltpu.SemaphoreType.DMA,
        ],
    )
    def gath(table_hbm, idx_hbm, out_hbm, idx_v, rows_v, sem):
        wid = lax.axis_index("s") * info.num_cores + lax.axis_index("c")
        base = wid * mw

        def body(i, carry):
            off = base + i * ch
            pltpu.sync_copy(idx_hbm.at[pl.ds(off, ch)], idx_v)
            pltpu.async_copy(table_hbm.at[idx_v], rows_v, sem).wait()
            pltpu.sync_copy(rows_v, out_hbm.at[pl.ds(off, ch)])
            return carry

        lax.fori_loop(0, mw // ch, body, 0)

    return gath(table, idxf)


def _full_spec(shape):
    nd = len(shape)
    return pl.BlockSpec(shape, lambda i, _nd=nd: (0,) * _nd)


def _x2_tile(g_ref, c_ref, w1f_ref, w1rf_ref, c1f_ref):
    g2 = g_ref[...].reshape(_GT * _K, _D)
    h1 = jnp.dot(g2, w1f_ref[...], preferred_element_type=jnp.float32,
                 precision=_HIGH)
    cc = jnp.dot(c_ref[...], w1rf_ref[...], preferred_element_type=jnp.float32,
                 precision=_HIGH)
    h3 = h1.reshape(_GT, _K, 64) + (c1f_ref[...] - cc)[:, None, :]
    return jnp.maximum(h3, 0.0).reshape(_GT * _K, 64)


def _acc_stats(sum_ref, ss_ref, h):
    @pl.when(pl.program_id(0) == 0)
    def _():
        sum_ref[...] = jnp.zeros_like(sum_ref)
        ss_ref[...] = jnp.zeros_like(ss_ref)

    sum_ref[...] += jnp.sum(h, axis=0, keepdims=True)
    ss_ref[...] += jnp.sum(h * h, axis=0, keepdims=True)


def _p1_body(g_ref, c_ref, w1p_ref, w1r_ref, b1_ref, sum_ref, ss_ref):
    g2 = g_ref[...].reshape(_GT * _K, _D)
    h1 = jnp.dot(g2, w1p_ref[...], preferred_element_type=jnp.float32,
                 precision=_HIGH)
    cc = jnp.dot(c_ref[...], w1r_ref[...], preferred_element_type=jnp.float32,
                 precision=_HIGH)
    h3 = h1.reshape(_GT, _K, 64) + (b1_ref[...] - cc)[:, None, :]
    _acc_stats(sum_ref, ss_ref, h3.reshape(_GT * _K, 64))


def _p2_body(g_ref, c_ref, w1f_ref, w1rf_ref, c1f_ref, w2t_ref, b2_ref,
             sum_ref, ss_ref):
    x2 = _x2_tile(g_ref, c_ref, w1f_ref, w1rf_ref, c1f_ref)
    h2 = jnp.dot(x2, w2t_ref[...], preferred_element_type=jnp.float32,
                 precision=_HIGH) + b2_ref[...]
    _acc_stats(sum_ref, ss_ref, h2)


def _p3_body(g_ref, c_ref, w1f_ref, w1rf_ref, c1f_ref, w2f_ref, c2f_ref,
             w3t_ref, b3_ref, sum_ref, ss_ref):
    x2 = _x2_tile(g_ref, c_ref, w1f_ref, w1rf_ref, c1f_ref)
    x3 = jnp.maximum(
        jnp.dot(x2, w2f_ref[...], preferred_element_type=jnp.float32,
                precision=_HIGH) + c2f_ref[...], 0.0)
    h3 = jnp.dot(x3, w3t_ref[...], preferred_element_type=jnp.float32,
                 precision=_HIGH) + b3_ref[...]
    _acc_stats(sum_ref, ss_ref, h3)


def _p4_body(g_ref, c_ref, w1f_ref, w1rf_ref, c1f_ref, w2f_ref, c2f_ref,
             w3f_ref, c3f_ref, out_ref):
    x2 = _x2_tile(g_ref, c_ref, w1f_ref, w1rf_ref, c1f_ref)
    x3 = jnp.maximum(
        jnp.dot(x2, w2f_ref[...], preferred_element_type=jnp.float32,
                precision=_HIGH) + c2f_ref[...], 0.0)
    y = jnp.maximum(
        jnp.dot(x3, w3f_ref[...], preferred_element_type=jnp.float32,
                precision=_HIGH) + c3f_ref[...], 0.0)
    out_ref[...] = jnp.max(y.reshape(_GT, _K, 128), axis=1)


def _run_pass(body, g3, cpos8, weights, out_ch, final=False):
    BP = g3.shape[0]
    grid = (BP // _GT,)
    in_specs = [
        pl.BlockSpec((_GT, _K, _D), lambda i: (i, 0, 0)),
        pl.BlockSpec((_GT, 8), lambda i: (i, 0)),
    ] + [_full_spec(w.shape) for w in weights]
    if final:
        out_specs = pl.BlockSpec((_GT, out_ch), lambda i: (i, 0))
        out_shape = jax.ShapeDtypeStruct((BP, out_ch), jnp.float32)
    else:
        out_specs = [pl.BlockSpec((1, out_ch), lambda i: (0, 0))] * 2
        out_shape = [jax.ShapeDtypeStruct((1, out_ch), jnp.float32)] * 2
    return pl.pallas_call(
        body, grid=grid, in_specs=in_specs,
        out_specs=out_specs, out_shape=out_shape,
    )(g3, cpos8, *weights)


def _fold(sum_v, ss_v, m_tot, b, g, be):
    mu = sum_v.reshape(-1) / m_tot
    var = ss_v.reshape(-1) / m_tot - mu * mu
    s = g / jnp.sqrt(var + _EPS)
    c = (b - mu) * s + be
    return s, c.reshape(1, -1)


def kernel(pos, feat, W1, b1, g1, be1, W2, b2, g2, be2, W3, b3, g3, be3):
    B, N, _ = pos.shape
    Df = feat.shape[-1]
    BP = B * _P
    M = BP * _K
    m_tot = jnp.float32(M)

    center_pos = _run_fps(pos)

    idxf = _run_knn(pos, center_pos)

    table = jnp.concatenate(
        [feat.reshape(B * N, Df), pos.reshape(B * N, 3),
         jnp.zeros((B * N, _D - Df - 3), jnp.float32)], axis=1)
    g_rows = _run_gather(table, idxf)
    g3d = g_rows.reshape(BP, _K, _D)

    cpos8 = jnp.concatenate(
        [center_pos.reshape(BP, 3), jnp.zeros((BP, 5), jnp.float32)], axis=1)

    w1p = jnp.zeros((_D, 64), jnp.float32)
    w1p = w1p.at[0:Df].set(W1[:, 3:].T).at[Df:Df + 3].set(W1[:, :3].T)
    w1r = jnp.zeros((8, 64), jnp.float32).at[0:3].set(W1[:, :3].T)
    w2t = W2.T
    w3t = W3.T
    b1r = b1.reshape(1, 64)
    b2r = b2.reshape(1, 64)
    b3r = b3.reshape(1, 128)

    sum1, ss1 = _run_pass(_p1_body, g3d, cpos8, [w1p, w1r, b1r], 64)
    s1, c1f = _fold(sum1, ss1, m_tot, b1, g1, be1)
    w1f = w1p * s1[None, :]
    w1rf = w1r * s1[None, :]

    sum2, ss2 = _run_pass(_p2_body, g3d, cpos8, [w1f, w1rf, c1f, w2t, b2r], 64)
    s2, c2f = _fold(sum2, ss2, m_tot, b2, g2, be2)
    w2f = w2t * s2[None, :]

    sum3, ss3 = _run_pass(_p3_body, g3d, cpos8,
                          [w1f, w1rf, c1f, w2f, c2f, w3t, b3r], 128)
    s3, c3f = _fold(sum3, ss3, m_tot, b3, g3, be3)
    w3f = w3t * s3[None, :]

    feat_res = _run_pass(_p4_body, g3d, cpos8,
                         [w1f, w1rf, c1f, w2f, c2f, w3f, c3f], 128,
                         final=True)
    return center_pos, feat_res.reshape(B, _P, 128)
